# baseline (speedup 1.0000x reference)
import functools

import jax
import jax.numpy as jnp
from jax import lax
from jax.experimental import pallas as pl
from jax.experimental.pallas import tpu as pltpu
from jax.experimental.pallas import tpu_sc as plsc

BATCH = 16384
DIM = 32
NC = 2
NS = 16
NW = NC * NS
BPW = BATCH // NW
CHUNK = 128
NCHUNK = BPW // CHUNK
GROUPS = BPW // 16


def _body(uidx_hbm, midx_hbm, mtab_hbm, utab_hbm, wb_hbm, bb_hbm, out_hbm,
          midx_v, uidx_v, mrows_v, urows_v, wb_v, bb_v, out_v, sem):
  wid = lax.axis_index("s") * NC + lax.axis_index("c")
  base = pl.multiple_of(wid * BPW, BPW)

  pltpu.sync_copy(midx_hbm.at[pl.ds(base, BPW)], midx_v)
  pltpu.sync_copy(uidx_hbm.at[pl.ds(base, BPW)], uidx_v)
  pltpu.sync_copy(wb_hbm, wb_v)
  pltpu.sync_copy(bb_hbm, bb_v)

  handles = []
  for j in range(NCHUNK):
    o = j * CHUNK
    handles.append(pltpu.async_copy(
        mtab_hbm.at[midx_v.at[pl.ds(o, CHUNK)]],
        mrows_v.at[pl.ds(o, CHUNK)], sem))
    handles.append(pltpu.async_copy(
        utab_hbm.at[uidx_v.at[pl.ds(o, CHUNK)]],
        urows_v.at[pl.ds(o, CHUNK)], sem))
  for h in handles:
    h.wait()

  lane = lax.iota(jnp.int32, 16)
  bvec = bb_v[...]

  def group(g, carry):
    row0 = g * 16
    rows = row0 + lane
    acc = bvec
    for d in range(DIM):
      cold = jnp.full((16,), d, dtype=jnp.int32)
      acc = acc + plsc.load_gather(mrows_v, [rows, cold]) * wb_v[d]
      acc = acc + plsc.load_gather(urows_v, [rows, cold]) * wb_v[DIM + d]
    out_v[pl.ds(pl.multiple_of(row0, 16), 16)] = acc
    return carry

  lax.fori_loop(0, GROUPS, group, 0)

  pltpu.sync_copy(out_v, out_hbm.at[pl.ds(base, BPW)])


@functools.partial(jax.jit, static_argnames=())
def _run(user_index, movie_index, movie_table, user_table, wb, bb):
  mesh = plsc.VectorSubcoreMesh(core_axis_name="c", subcore_axis_name="s")
  return pl.kernel(
      _body,
      out_type=jax.ShapeDtypeStruct((BATCH,), jnp.float32),
      mesh=mesh,
      scratch_types=[
          pltpu.VMEM((BPW,), jnp.int32),
          pltpu.VMEM((BPW,), jnp.int32),
          pltpu.VMEM((BPW, DIM), jnp.float32),
          pltpu.VMEM((BPW, DIM), jnp.float32),
          pltpu.VMEM((2 * DIM, 16), jnp.float32),
          pltpu.VMEM((16,), jnp.float32),
          pltpu.VMEM((BPW,), jnp.float32),
          pltpu.SemaphoreType.DMA,
      ],
      compiler_params=pltpu.CompilerParams(
          needs_layout_passes=False, use_tc_tiling_on_sc=False),
  )(user_index, movie_index, movie_table, user_table, wb, bb)


def kernel(user_index, movie_index, movie_table, user_table, W, b):
  wb = jnp.broadcast_to(W.reshape(2 * DIM, 1), (2 * DIM, 16))
  bb = jnp.broadcast_to(b.reshape(1), (16,)).astype(jnp.float32)
  return _run(user_index.astype(jnp.int32), movie_index.astype(jnp.int32),
              movie_table, user_table, wb, bb)


# speedup vs baseline: 5.8455x; 5.8455x over previous
import functools

import jax
import jax.numpy as jnp
from jax import lax
from jax.experimental import pallas as pl
from jax.experimental.pallas import tpu as pltpu
from jax.experimental.pallas import tpu_sc as plsc

BATCH = 16384
DIM = 32
NROWS = 1000000
NC = 2
NS = 16
NW = NC * NS
BPW = BATCH // NW
CHUNK = 128
NCHUNK = BPW // CHUNK

BLK = 8192
NBLK = (NROWS + BLK - 1) // BLK


def _matvec_body(tm_ref, tu_ref, wm_ref, wu_ref, sm_ref, su_ref):
  sm_ref[...] = jnp.sum(tm_ref[...] * wm_ref[...], axis=0)
  su_ref[...] = jnp.sum(tu_ref[...] * wu_ref[...], axis=0)


def _matvec(tm, tu, wm, wu):
  return pl.pallas_call(
      _matvec_body,
      grid=(NBLK,),
      in_specs=[
          pl.BlockSpec((DIM, BLK), lambda i: (0, i)),
          pl.BlockSpec((DIM, BLK), lambda i: (0, i)),
          pl.BlockSpec((DIM, 1), lambda i: (0, 0)),
          pl.BlockSpec((DIM, 1), lambda i: (0, 0)),
      ],
      out_specs=[
          pl.BlockSpec((BLK,), lambda i: (i,)),
          pl.BlockSpec((BLK,), lambda i: (i,)),
      ],
      out_shape=[
          jax.ShapeDtypeStruct((NROWS,), jnp.float32),
          jax.ShapeDtypeStruct((NROWS,), jnp.float32),
      ],
  )(tm, tu, wm, wu)


def _sc_body(midx_hbm, uidx_hbm, sm_hbm, su_hbm, bb_hbm, out_hbm,
             midx_v, uidx_v, sm_v, su_v, bb_v, out_v, sem):
  wid = lax.axis_index("s") * NC + lax.axis_index("c")
  base = pl.multiple_of(wid * BPW, BPW)

  pltpu.sync_copy(midx_hbm.at[pl.ds(base, BPW)], midx_v)
  pltpu.sync_copy(uidx_hbm.at[pl.ds(base, BPW)], uidx_v)
  pltpu.sync_copy(bb_hbm, bb_v)

  handles = []
  for j in range(NCHUNK):
    o = j * CHUNK
    handles.append(pltpu.async_copy(
        sm_hbm.at[midx_v.at[pl.ds(o, CHUNK)]], sm_v.at[pl.ds(o, CHUNK)], sem))
    handles.append(pltpu.async_copy(
        su_hbm.at[uidx_v.at[pl.ds(o, CHUNK)]], su_v.at[pl.ds(o, CHUNK)], sem))
  for h in handles:
    h.wait()

  bvec = bb_v[...]
  for k in range(BPW // 16):
    o = k * 16
    out_v[pl.ds(o, 16)] = sm_v[pl.ds(o, 16)] + su_v[pl.ds(o, 16)] + bvec

  pltpu.sync_copy(out_v, out_hbm.at[pl.ds(base, BPW)])


def _sc_gather_add(midx, uidx, sm, su, bb):
  mesh = plsc.VectorSubcoreMesh(core_axis_name="c", subcore_axis_name="s")
  return pl.kernel(
      _sc_body,
      out_type=jax.ShapeDtypeStruct((BATCH,), jnp.float32),
      mesh=mesh,
      scratch_types=[
          pltpu.VMEM((BPW,), jnp.int32),
          pltpu.VMEM((BPW,), jnp.int32),
          pltpu.VMEM((BPW,), jnp.float32),
          pltpu.VMEM((BPW,), jnp.float32),
          pltpu.VMEM((16,), jnp.float32),
          pltpu.VMEM((BPW,), jnp.float32),
          pltpu.SemaphoreType.DMA,
      ],
  )(midx, uidx, sm, su, bb)


def kernel(user_index, movie_index, movie_table, user_table, W, b):
  tm = movie_table.T
  tu = user_table.T
  wm = W[:DIM].reshape(DIM, 1)
  wu = W[DIM:].reshape(DIM, 1)
  bb = jnp.broadcast_to(b.reshape(1), (16,)).astype(jnp.float32)
  sm, su = _matvec(tm, tu, wm, wu)
  return _sc_gather_add(movie_index.astype(jnp.int32),
                        user_index.astype(jnp.int32), sm, su, bb)


# speedup vs baseline: 6.0995x; 1.0435x over previous
import functools

import jax
import jax.numpy as jnp
from jax import lax
from jax.experimental import pallas as pl
from jax.experimental.pallas import tpu as pltpu
from jax.experimental.pallas import tpu_sc as plsc

BATCH = 16384
DIM = 32
NROWS = 1000000
NC = 2
NS = 16
NW = NC * NS
BPW = BATCH // NW
CHUNK = 128
NCHUNK = BPW // CHUNK

BLK = 8192
NBLK = (NROWS + BLK - 1) // BLK


def _matvec_body(tm_ref, tu_ref, wm_ref, wu_ref, sm_ref, su_ref):
  sm_ref[...] = jnp.dot(wm_ref[...], tm_ref[...],
                        preferred_element_type=jnp.float32)[0]
  su_ref[...] = jnp.dot(wu_ref[...], tu_ref[...],
                        preferred_element_type=jnp.float32)[0]


def _matvec(tm, tu, wm, wu):
  return pl.pallas_call(
      _matvec_body,
      grid=(NBLK,),
      in_specs=[
          pl.BlockSpec((DIM, BLK), lambda i: (0, i)),
          pl.BlockSpec((DIM, BLK), lambda i: (0, i)),
          pl.BlockSpec((8, DIM), lambda i: (0, 0)),
          pl.BlockSpec((8, DIM), lambda i: (0, 0)),
      ],
      out_specs=[
          pl.BlockSpec((BLK,), lambda i: (i,)),
          pl.BlockSpec((BLK,), lambda i: (i,)),
      ],
      out_shape=[
          jax.ShapeDtypeStruct((NROWS,), jnp.float32),
          jax.ShapeDtypeStruct((NROWS,), jnp.float32),
      ],
  )(tm, tu, wm, wu)


def _sc_body(midx_hbm, uidx_hbm, sm_hbm, su_hbm, bb_hbm, out_hbm,
             midx_v, uidx_v, sm_v, su_v, bb_v, out_v, sem):
  wid = lax.axis_index("s") * NC + lax.axis_index("c")
  base = pl.multiple_of(wid * BPW, BPW)

  pltpu.sync_copy(midx_hbm.at[pl.ds(base, BPW)], midx_v)
  pltpu.sync_copy(uidx_hbm.at[pl.ds(base, BPW)], uidx_v)
  pltpu.sync_copy(bb_hbm, bb_v)

  handles = []
  for j in range(NCHUNK):
    o = j * CHUNK
    handles.append(pltpu.async_copy(
        sm_hbm.at[midx_v.at[pl.ds(o, CHUNK)]], sm_v.at[pl.ds(o, CHUNK)], sem))
    handles.append(pltpu.async_copy(
        su_hbm.at[uidx_v.at[pl.ds(o, CHUNK)]], su_v.at[pl.ds(o, CHUNK)], sem))
  for h in handles:
    h.wait()

  bvec = bb_v[...]
  for k in range(BPW // 16):
    o = k * 16
    out_v[pl.ds(o, 16)] = sm_v[pl.ds(o, 16)] + su_v[pl.ds(o, 16)] + bvec

  pltpu.sync_copy(out_v, out_hbm.at[pl.ds(base, BPW)])


def _sc_gather_add(midx, uidx, sm, su, bb):
  mesh = plsc.VectorSubcoreMesh(core_axis_name="c", subcore_axis_name="s")
  return pl.kernel(
      _sc_body,
      out_type=jax.ShapeDtypeStruct((BATCH,), jnp.float32),
      mesh=mesh,
      scratch_types=[
          pltpu.VMEM((BPW,), jnp.int32),
          pltpu.VMEM((BPW,), jnp.int32),
          pltpu.VMEM((BPW,), jnp.float32),
          pltpu.VMEM((BPW,), jnp.float32),
          pltpu.VMEM((16,), jnp.float32),
          pltpu.VMEM((BPW,), jnp.float32),
          pltpu.SemaphoreType.DMA,
      ],
  )(midx, uidx, sm, su, bb)


def kernel(user_index, movie_index, movie_table, user_table, W, b):
  tm = movie_table.T
  tu = user_table.T
  wm = jnp.zeros((8, DIM), jnp.float32).at[0].set(W[:DIM, 0])
  wu = jnp.zeros((8, DIM), jnp.float32).at[0].set(W[DIM:, 0])
  bb = jnp.broadcast_to(b.reshape(1), (16,)).astype(jnp.float32)
  sm, su = _matvec(tm, tu, wm, wu)
  return _sc_gather_add(movie_index.astype(jnp.int32),
                        user_index.astype(jnp.int32), sm, su, bb)


# speedup vs baseline: 8.0566x; 1.3209x over previous
import functools

import jax
import jax.numpy as jnp
from jax import lax
from jax.experimental import pallas as pl
from jax.experimental.pallas import tpu as pltpu
from jax.experimental.pallas import tpu_sc as plsc

BATCH = 16384
DIM = 32
NROWS = 1000000
NC = 2
NS = 16
NW = NC * NS
BPW = BATCH // NW
CHUNK = 128
NCHUNK = BPW // CHUNK

BLK = 16384
NBLK = (NROWS + BLK - 1) // BLK


def _matvec_body(tm_ref, tu_ref, wm_ref, wu_ref, sm_ref, su_ref):
  sm_ref[...] = jnp.dot(wm_ref[...], tm_ref[...],
                        preferred_element_type=jnp.float32)[0]
  su_ref[...] = jnp.dot(wu_ref[...], tu_ref[...],
                        preferred_element_type=jnp.float32)[0]


def _matvec(tm, tu, wm, wu):
  return pl.pallas_call(
      _matvec_body,
      grid=(NBLK,),
      in_specs=[
          pl.BlockSpec((DIM, BLK), lambda i: (0, i)),
          pl.BlockSpec((DIM, BLK), lambda i: (0, i)),
          pl.BlockSpec((8, DIM), lambda i: (0, 0)),
          pl.BlockSpec((8, DIM), lambda i: (0, 0)),
      ],
      out_specs=[
          pl.BlockSpec((BLK,), lambda i: (i,)),
          pl.BlockSpec((BLK,), lambda i: (i,)),
      ],
      out_shape=[
          jax.ShapeDtypeStruct((NROWS,), jnp.float32),
          jax.ShapeDtypeStruct((NROWS,), jnp.float32),
      ],
  )(tm, tu, wm, wu)


def _sc_body(midx_hbm, uidx_hbm, sm_hbm, su_hbm, bb_hbm, out_hbm,
             midx_v, uidx_v, sm_v, su_v, bb_v, out_v, sem):
  wid = lax.axis_index("s") * NC + lax.axis_index("c")
  base = pl.multiple_of(wid * BPW, BPW)

  pltpu.sync_copy(midx_hbm.at[pl.ds(base, BPW)], midx_v)
  pltpu.sync_copy(uidx_hbm.at[pl.ds(base, BPW)], uidx_v)
  pltpu.sync_copy(bb_hbm, bb_v)

  handles = []
  for j in range(NCHUNK):
    o = j * CHUNK
    handles.append(pltpu.async_copy(
        sm_hbm.at[midx_v.at[pl.ds(o, CHUNK)]], sm_v.at[pl.ds(o, CHUNK)], sem))
    handles.append(pltpu.async_copy(
        su_hbm.at[uidx_v.at[pl.ds(o, CHUNK)]], su_v.at[pl.ds(o, CHUNK)], sem))
  for h in handles:
    h.wait()

  bvec = bb_v[...]
  for k in range(BPW // 16):
    o = k * 16
    out_v[pl.ds(o, 16)] = sm_v[pl.ds(o, 16)] + su_v[pl.ds(o, 16)] + bvec

  pltpu.sync_copy(out_v, out_hbm.at[pl.ds(base, BPW)])


def _sc_gather_add(midx, uidx, sm, su, bb):
  mesh = plsc.VectorSubcoreMesh(core_axis_name="c", subcore_axis_name="s")
  return pl.kernel(
      _sc_body,
      out_type=jax.ShapeDtypeStruct((BATCH,), jnp.float32),
      mesh=mesh,
      scratch_types=[
          pltpu.VMEM((BPW,), jnp.int32),
          pltpu.VMEM((BPW,), jnp.int32),
          pltpu.VMEM((BPW,), jnp.float32),
          pltpu.VMEM((BPW,), jnp.float32),
          pltpu.VMEM((16,), jnp.float32),
          pltpu.VMEM((BPW,), jnp.float32),
          pltpu.SemaphoreType.DMA,
      ],
  )(midx, uidx, sm, su, bb)


def kernel(user_index, movie_index, movie_table, user_table, W, b):
  tm = movie_table.T
  tu = user_table.T
  wm = jnp.zeros((8, DIM), jnp.float32).at[0].set(W[:DIM, 0])
  wu = jnp.zeros((8, DIM), jnp.float32).at[0].set(W[DIM:, 0])
  bb = jnp.broadcast_to(b.reshape(1), (16,)).astype(jnp.float32)
  sm, su = _matvec(tm, tu, wm, wu)
  return _sc_gather_add(movie_index.astype(jnp.int32),
                        user_index.astype(jnp.int32), sm, su, bb)


# speedup vs baseline: 8.8659x; 1.1004x over previous
import functools

import jax
import jax.numpy as jnp
from jax import lax
from jax.experimental import pallas as pl
from jax.experimental.pallas import tpu as pltpu
from jax.experimental.pallas import tpu_sc as plsc

BATCH = 16384
DIM = 32
NROWS = 1000000
NC = 2
NS = 16
NW = NC * NS
BPW = BATCH // NW
CHUNK = 128
NCHUNK = BPW // CHUNK

BLK = 32768
NBLK = (NROWS + BLK - 1) // BLK


def _matvec_body(tm_ref, tu_ref, wm_ref, wu_ref, sm_ref, su_ref):
  sm_ref[...] = jnp.dot(wm_ref[...], tm_ref[...],
                        preferred_element_type=jnp.float32)[0]
  su_ref[...] = jnp.dot(wu_ref[...], tu_ref[...],
                        preferred_element_type=jnp.float32)[0]


def _matvec(tm, tu, wm, wu):
  return pl.pallas_call(
      _matvec_body,
      grid=(NBLK,),
      in_specs=[
          pl.BlockSpec((DIM, BLK), lambda i: (0, i)),
          pl.BlockSpec((DIM, BLK), lambda i: (0, i)),
          pl.BlockSpec((8, DIM), lambda i: (0, 0)),
          pl.BlockSpec((8, DIM), lambda i: (0, 0)),
      ],
      out_specs=[
          pl.BlockSpec((BLK,), lambda i: (i,)),
          pl.BlockSpec((BLK,), lambda i: (i,)),
      ],
      out_shape=[
          jax.ShapeDtypeStruct((NROWS,), jnp.float32),
          jax.ShapeDtypeStruct((NROWS,), jnp.float32),
      ],
  )(tm, tu, wm, wu)


def _sc_body(midx_hbm, uidx_hbm, sm_hbm, su_hbm, bb_hbm, out_hbm,
             midx_v, uidx_v, sm_v, su_v, bb_v, out_v, sem):
  wid = lax.axis_index("s") * NC + lax.axis_index("c")
  base = pl.multiple_of(wid * BPW, BPW)

  pltpu.sync_copy(midx_hbm.at[pl.ds(base, BPW)], midx_v)
  pltpu.sync_copy(uidx_hbm.at[pl.ds(base, BPW)], uidx_v)
  pltpu.sync_copy(bb_hbm, bb_v)

  handles = []
  for j in range(NCHUNK):
    o = j * CHUNK
    handles.append(pltpu.async_copy(
        sm_hbm.at[midx_v.at[pl.ds(o, CHUNK)]], sm_v.at[pl.ds(o, CHUNK)], sem))
    handles.append(pltpu.async_copy(
        su_hbm.at[uidx_v.at[pl.ds(o, CHUNK)]], su_v.at[pl.ds(o, CHUNK)], sem))
  for h in handles:
    h.wait()

  bvec = bb_v[...]
  for k in range(BPW // 16):
    o = k * 16
    out_v[pl.ds(o, 16)] = sm_v[pl.ds(o, 16)] + su_v[pl.ds(o, 16)] + bvec

  pltpu.sync_copy(out_v, out_hbm.at[pl.ds(base, BPW)])


def _sc_gather_add(midx, uidx, sm, su, bb):
  mesh = plsc.VectorSubcoreMesh(core_axis_name="c", subcore_axis_name="s")
  return pl.kernel(
      _sc_body,
      out_type=jax.ShapeDtypeStruct((BATCH,), jnp.float32),
      mesh=mesh,
      scratch_types=[
          pltpu.VMEM((BPW,), jnp.int32),
          pltpu.VMEM((BPW,), jnp.int32),
          pltpu.VMEM((BPW,), jnp.float32),
          pltpu.VMEM((BPW,), jnp.float32),
          pltpu.VMEM((16,), jnp.float32),
          pltpu.VMEM((BPW,), jnp.float32),
          pltpu.SemaphoreType.DMA,
      ],
  )(midx, uidx, sm, su, bb)


def kernel(user_index, movie_index, movie_table, user_table, W, b):
  tm = movie_table.T
  tu = user_table.T
  wm = jnp.zeros((8, DIM), jnp.float32).at[0].set(W[:DIM, 0])
  wu = jnp.zeros((8, DIM), jnp.float32).at[0].set(W[DIM:, 0])
  bb = jnp.broadcast_to(b.reshape(1), (16,)).astype(jnp.float32)
  sm, su = _matvec(tm, tu, wm, wu)
  return _sc_gather_add(movie_index.astype(jnp.int32),
                        user_index.astype(jnp.int32), sm, su, bb)
